# baseline (speedup 1.0000x reference)
import functools

import numpy as np

import jax
import jax.numpy as jnp
from jax import lax
from jax.experimental import pallas as pl
from jax.experimental.pallas import tpu as pltpu
from jax.experimental.pallas import tpu_sc as plsc

NUM_GROUPS = 6
GROUP_SIZE = 16
OUT_DIM = 64
F_IN = NUM_GROUPS * GROUP_SIZE
F_OUT = NUM_GROUPS * OUT_DIM
GSTRIDE = 6320
SEG_OFF = (0, 1504, 1808, 2312, 3312, 4312, 5312)
SEG_N = (1500, 300, 500, 1000, 1000, 1000, 1000)
TABLE_ROWS = NUM_GROUPS * GSTRIDE

NC, NS, LANES = 2, 16, 16
NW = NC * NS
C = 20
SLOTS = 7
IDX_PER_REF = SLOTS * C
ROWS_PER_CHUNK = C * NUM_GROUPS * SLOTS


def _bf16_bits(y):
    ui = lax.bitcast_convert_type(y, jnp.int32)
    r = ui + 0x7FFF + ((ui >> 16) & 1)
    return (r >> 16) & 0xFFFF


def _proj_body(sp_ref, ab_ref, it_ref, mv_ref, w_ref, out_ref):
    wg = w_ref[0]

    def proj(tab, c0, c1, r0, n):
        y = lax.dot_general(
            tab, wg[:, c0:c1], (((1,), (1,)), ((), ())),
            preferred_element_type=jnp.float32,
            precision=lax.Precision.HIGHEST)
        out_ref[pl.ds(r0, n), :] = (
            _bf16_bits(y[:, :32]) | (_bf16_bits(y[:, 32:]) << 16))

    proj(sp_ref[...], 0, 32, 0, 1504)
    proj(ab_ref[...], 32, 48, 1504, 304)
    proj(it_ref[...], 48, 64, 1808, 504)
    for j in range(4):
        proj(mv_ref[...], 64 + 16 * j, 80 + 16 * j, 2312 + 1000 * j, 1000)
    out_ref[pl.ds(6312, 8), :] = jnp.zeros((8, OUT_DIM // 2), jnp.int32)


def _build_flat_table(sp_p, ab_p, it_p, mv, w_pad):
    full = lambda shape: pl.BlockSpec(shape, lambda g: (0,) * len(shape))
    return pl.pallas_call(
        _proj_body,
        grid=(NUM_GROUPS,),
        in_specs=[
            full((1504, 32)), full((304, 16)), full((504, 16)), full((1000, 16)),
            pl.BlockSpec((1, 64, 144), lambda g: (g, 0, 0)),
        ],
        out_specs=pl.BlockSpec((GSTRIDE, OUT_DIM // 2), lambda g: (g, 0)),
        out_shape=jax.ShapeDtypeStruct((TABLE_ROWS, OUT_DIM // 2), jnp.int32),
    )(sp_p, ab_p, it_p, mv, w_pad)


def _ynum_body(x_ref, bd_ref, b_ref, out_ref):
    bt, s, f = x_ref.shape
    y = lax.dot_general(
        x_ref[...].reshape(bt * s, f), bd_ref[...], (((1,), (0,)), ((), ())),
        preferred_element_type=jnp.float32,
        precision=lax.Precision.HIGHEST) + b_ref[...]
    packed = jnp.concatenate(
        [_bf16_bits(y[:, g * OUT_DIM:g * OUT_DIM + 32])
         | (_bf16_bits(y[:, g * OUT_DIM + 32:(g + 1) * OUT_DIM]) << 16)
         for g in range(NUM_GROUPS)], axis=1)
    out_ref[...] = packed.reshape(bt, s, F_OUT // 2)


def _build_ynum(x, bd, b_row):
    n, s, f = x.shape
    bt = 256
    return pl.pallas_call(
        _ynum_body,
        grid=(n // bt,),
        in_specs=[
            pl.BlockSpec((bt, s, f), lambda i: (i, 0, 0)),
            pl.BlockSpec((F_IN, F_OUT), lambda i: (0, 0)),
            pl.BlockSpec((1, F_OUT), lambda i: (0, 0)),
        ],
        out_specs=pl.BlockSpec((bt, s, F_OUT // 2), lambda i: (i, 0, 0)),
        out_shape=jax.ShapeDtypeStruct((n, s, F_OUT // 2), jnp.int32),
    )(x, bd, b_row)


def _sc_encode(flat_table, x, ynum, consts):
    n_rows, s_len = x.shape[0], x.shape[1]
    rpw = n_rows // NW
    n_chunks = rpw

    mesh = plsc.VectorSubcoreMesh(core_axis_name="c", subcore_axis_name="s")

    @functools.partial(
        pl.kernel, mesh=mesh,
        compiler_params=pltpu.CompilerParams(
            needs_layout_passes=False, use_tc_tiling_on_sc=False),
        out_type=jax.ShapeDtypeStruct((n_rows, s_len, F_OUT), jnp.float32),
        scratch_types=(
            [pltpu.VMEM((2, 1, C, F_IN), jnp.float32),
             pltpu.VMEM((2, NUM_GROUPS, IDX_PER_REF), jnp.int32),
             pltpu.VMEM((2, ROWS_PER_CHUNK, OUT_DIM // 2), jnp.int32),
             pltpu.VMEM((2, 1, C, F_OUT // 2), jnp.int32),
             pltpu.VMEM((2, 1, C, F_OUT), jnp.float32),
             pltpu.VMEM((1 + NUM_GROUPS, LANES), jnp.int32)]
            + [pltpu.SemaphoreType.DMA((2,))] * 4),
    )
    def run(table_hbm, x_hbm, ynum_hbm, consts_hbm, out_hbm,
            x_v, idx_v, rows_v, yp_v, out_v, cv_v, xsem, gsem, ysem, osem):
        xsems = lambda p: xsem.at[p]
        gsems = lambda p: gsem.at[p]
        ysems = lambda p: ysem.at[p]
        osems = lambda p: osem.at[p]
        wid = lax.axis_index("s") * NC + lax.axis_index("c")
        w0 = wid * rpw
        pltpu.sync_copy(consts_hbm, cv_v)
        lane = lax.broadcasted_iota(jnp.int32, (LANES,), 0)
        segmax = cv_v[0]
        off_vecs = [cv_v[1 + g] for g in range(NUM_GROUPS)]
        slot_mask = lane < SLOTS

        def fetch_x(i, p):
            return pltpu.make_async_copy(
                x_hbm.at[pl.ds(w0 + i, 1)], x_v.at[p], xsems(p))

        def compute_idx(p):
            def idx_body(t, _):
                for g in range(NUM_GROUPS):
                    xg = x_v[p, 0, t, pl.ds(g * GROUP_SIZE, GROUP_SIZE)]
                    iv = jnp.minimum(jnp.maximum(xg.astype(jnp.int32), 0), segmax)
                    plsc.store_scatter(idx_v,
                                       [jnp.full((LANES,), p, jnp.int32),
                                        jnp.full((LANES,), g, jnp.int32),
                                        lane * C + t],
                                       iv + off_vecs[g], mask=slot_mask)
                return 0
            lax.fori_loop(0, C, idx_body, 0)

        def gather_cps(p):
            return [pltpu.make_async_copy(
                        table_hbm.at[idx_v.at[p, g]],
                        rows_v.at[p, pl.ds(g * IDX_PER_REF, IDX_PER_REF)],
                        gsems(p))
                    for g in range(NUM_GROUPS)]

        def fire_chunk(i, p):
            for cp in gather_cps(p):
                cp.start()
            pltpu.make_async_copy(
                ynum_hbm.at[pl.ds(w0 + i, 1)], yp_v.at[p], ysems(p)).start()

        def wait_chunk(p):
            for cp in gather_cps(p):
                cp.wait()
            pltpu.make_async_copy(
                ynum_hbm.at[pl.ds(0, 1)], yp_v.at[p], ysems(p)).wait()

        def compute_chunk(p):
            def tok_body(t, _):
                for g in range(NUM_GROUPS):
                    y0 = yp_v[p, 0, t, pl.ds(g * 32, LANES)]
                    y1 = yp_v[p, 0, t, pl.ds(g * 32 + LANES, LANES)]
                    acc = [lax.bitcast_convert_type(y0 << 16, jnp.float32),
                           lax.bitcast_convert_type(y1 << 16, jnp.float32),
                           lax.bitcast_convert_type(
                               y0 & jnp.int32(-65536), jnp.float32),
                           lax.bitcast_convert_type(
                               y1 & jnp.int32(-65536), jnp.float32)]
                    for s in range(SLOTS):
                        r = (g * SLOTS + s) * C + t
                        for h in range(2):
                            v = rows_v[p, r, pl.ds(h * LANES, LANES)]
                            acc[h] = acc[h] + lax.bitcast_convert_type(
                                v << 16, jnp.float32)
                            acc[2 + h] = acc[2 + h] + lax.bitcast_convert_type(
                                v & jnp.int32(-65536), jnp.float32)
                    for c in range(4):
                        out_v[p, 0, t, pl.ds(g * OUT_DIM + c * LANES, LANES)] = (
                            jnp.maximum(acc[c], 0.0))
                return 0
            lax.fori_loop(0, C, tok_body, 0)

        def flush_chunk(i, p):
            pltpu.make_async_copy(out_v.at[p], out_hbm.at[pl.ds(w0 + i, 1)],
                                  osems(p)).start()

        def drain_out(p):
            pltpu.make_async_copy(out_v.at[p], out_hbm.at[pl.ds(0, 1)],
                                  osems(p)).wait()

        cp0 = fetch_x(0, 0)
        cp0.start()
        cp0.wait()
        compute_idx(0)
        fire_chunk(0, 0)
        fetch_x(1, 1).start()

        def loop_body(i, _):
            p = lax.rem(i, 2)
            pltpu.make_async_copy(
                x_hbm.at[pl.ds(0, 1)], x_v.at[p], xsems(p)).wait()
            compute_idx(p)

            @pl.when(i >= 2)
            def _():
                drain_out(p)
            fire_chunk(i, p)

            @pl.when(i + 1 < n_chunks)
            def _():
                fetch_x(i + 1, 1 - p).start()

            wait_chunk(1 - p)
            compute_chunk(1 - p)
            flush_chunk(i - 1, 1 - p)
            return 0

        lax.fori_loop(1, n_chunks, loop_body, 0)

        pl_last = (n_chunks - 1) % 2
        wait_chunk(pl_last)
        compute_chunk(pl_last)
        flush_chunk(n_chunks - 1, pl_last)
        drain_out(0)
        drain_out(1)

    return run(flat_table, x, ynum, consts)


def _make_consts():
    consts = np.zeros((1 + NUM_GROUPS, LANES), np.int32)
    consts[0, :SLOTS] = [n - 1 for n in SEG_N]
    for g in range(NUM_GROUPS):
        consts[1 + g, :SLOTS] = [g * GSTRIDE + o for o in SEG_OFF]
    return jnp.asarray(consts)


def _make_bd_selector():
    s = np.zeros((F_IN, NUM_GROUPS, 9), np.float32)
    for g in range(NUM_GROUPS):
        for k in range(9):
            s[g * GROUP_SIZE + 7 + k, g, k] = 1.0
    return jnp.asarray(s)


def kernel(x, ability_emb, item_emb, species_emb, move_emb, W, b):
    sp_p = jnp.pad(species_emb, ((0, 4), (0, 0)))
    ab_p = jnp.pad(ability_emb, ((0, 4), (0, 0)))
    it_p = jnp.pad(item_emb, ((0, 4), (0, 0)))
    w_pad = jnp.pad(W, ((0, 0), (0, 0), (0, 7)))

    bd = jnp.einsum("fgk,gok->fgo", _make_bd_selector(),
                    W[:, :, 128:137]).reshape(F_IN, F_OUT)

    flat_table = _build_flat_table(sp_p, ab_p, it_p, move_emb, w_pad)
    ynum = _build_ynum(x, bd, b.reshape(1, F_OUT))
    return _sc_encode(flat_table, x, ynum, _make_consts())


# speedup vs baseline: 1.0526x; 1.0526x over previous
import functools

import numpy as np

import jax
import jax.numpy as jnp
from jax import lax
from jax.experimental import pallas as pl
from jax.experimental.pallas import tpu as pltpu
from jax.experimental.pallas import tpu_sc as plsc

NUM_GROUPS = 6
GROUP_SIZE = 16
OUT_DIM = 64
F_IN = NUM_GROUPS * GROUP_SIZE
F_OUT = NUM_GROUPS * OUT_DIM
GSTRIDE = 6320
SEG_OFF = (0, 1504, 1808, 2312, 3312, 4312, 5312)
SEG_N = (1500, 300, 500, 1000, 1000, 1000, 1000)
TABLE_ROWS = NUM_GROUPS * GSTRIDE

NC, NS, LANES = 2, 16, 16
NW = NC * NS
C = 16
SLOTS = 7
IDX_PER_REF = SLOTS * C
ROWS_PER_CHUNK = C * NUM_GROUPS * SLOTS


def _bf16_bits(y):
    ui = lax.bitcast_convert_type(y, jnp.int32)
    r = ui + 0x7FFF + ((ui >> 16) & 1)
    return (r >> 16) & 0xFFFF


def _proj_body(sp_ref, ab_ref, it_ref, mv_ref, w_ref, out_ref):
    wg = w_ref[0]

    def proj(tab, c0, c1, r0, n):
        y = lax.dot_general(
            tab, wg[:, c0:c1], (((1,), (1,)), ((), ())),
            preferred_element_type=jnp.float32,
            precision=lax.Precision.HIGHEST)
        out_ref[pl.ds(r0, n), :] = (
            _bf16_bits(y[:, :32]) | (_bf16_bits(y[:, 32:]) << 16))

    proj(sp_ref[...], 0, 32, 0, 1504)
    proj(ab_ref[...], 32, 48, 1504, 304)
    proj(it_ref[...], 48, 64, 1808, 504)
    for j in range(4):
        proj(mv_ref[...], 64 + 16 * j, 80 + 16 * j, 2312 + 1000 * j, 1000)
    out_ref[pl.ds(6312, 8), :] = jnp.zeros((8, OUT_DIM // 2), jnp.int32)


def _build_flat_table(sp_p, ab_p, it_p, mv, w_pad):
    full = lambda shape: pl.BlockSpec(shape, lambda g: (0,) * len(shape))
    return pl.pallas_call(
        _proj_body,
        grid=(NUM_GROUPS,),
        in_specs=[
            full((1504, 32)), full((304, 16)), full((504, 16)), full((1000, 16)),
            pl.BlockSpec((1, 64, 144), lambda g: (g, 0, 0)),
        ],
        out_specs=pl.BlockSpec((GSTRIDE, OUT_DIM // 2), lambda g: (g, 0)),
        out_shape=jax.ShapeDtypeStruct((TABLE_ROWS, OUT_DIM // 2), jnp.int32),
    )(sp_p, ab_p, it_p, mv, w_pad)


def _ynum_body(x_ref, bd_ref, b_ref, out_ref):
    y = lax.dot_general(
        x_ref[...].astype(jnp.bfloat16), bd_ref[...].astype(jnp.bfloat16),
        (((1,), (0,)), ((), ())),
        preferred_element_type=jnp.float32) + b_ref[...]
    out_ref[...] = jnp.concatenate(
        [_bf16_bits(y[:, g * OUT_DIM:g * OUT_DIM + 32])
         | (_bf16_bits(y[:, g * OUT_DIM + 32:(g + 1) * OUT_DIM]) << 16)
         for g in range(NUM_GROUPS)], axis=1)


def _build_ynum(xf, bd, b_row):
    n = xf.shape[0]
    bt = 4096
    return pl.pallas_call(
        _ynum_body,
        grid=(n // bt,),
        in_specs=[
            pl.BlockSpec((bt, F_IN), lambda i: (i, 0)),
            pl.BlockSpec((F_IN, F_OUT), lambda i: (0, 0)),
            pl.BlockSpec((1, F_OUT), lambda i: (0, 0)),
        ],
        out_specs=pl.BlockSpec((bt, F_OUT // 2), lambda i: (i, 0)),
        out_shape=jax.ShapeDtypeStruct((n, F_OUT // 2), jnp.int32),
    )(xf, bd, b_row)


def _sc_encode(flat_table, xf, ynum, consts):
    n_tokens = xf.shape[0]
    tpw = n_tokens // NW
    n_chunks = tpw // C

    mesh = plsc.VectorSubcoreMesh(core_axis_name="c", subcore_axis_name="s")

    @functools.partial(
        pl.kernel, mesh=mesh,
        compiler_params=pltpu.CompilerParams(
            needs_layout_passes=False, use_tc_tiling_on_sc=False),
        out_type=jax.ShapeDtypeStruct((n_tokens, F_OUT), jnp.float32),
        scratch_types=(
            [pltpu.VMEM((2, C, F_IN), jnp.float32),
             pltpu.VMEM((2, NUM_GROUPS, IDX_PER_REF), jnp.int32),
             pltpu.VMEM((2, ROWS_PER_CHUNK, OUT_DIM // 2), jnp.int32),
             pltpu.VMEM((2, C, F_OUT // 2), jnp.int32),
             pltpu.VMEM((2, C, F_OUT), jnp.float32),
             pltpu.VMEM((1 + NUM_GROUPS, LANES), jnp.int32)]
            + [pltpu.SemaphoreType.DMA((2,))] * 4),
    )
    def run(table_hbm, xf_hbm, ynum_hbm, consts_hbm, out_hbm,
            x_v, idx_v, rows_v, yp_v, out_v, cv_v, xsem, gsem, ysem, osem):
        xsems = lambda p: xsem.at[p]
        gsems = lambda p: gsem.at[p]
        ysems = lambda p: ysem.at[p]
        osems = lambda p: osem.at[p]
        wid = lax.axis_index("s") * NC + lax.axis_index("c")
        w0 = wid * tpw
        pltpu.sync_copy(consts_hbm, cv_v)
        lane = lax.broadcasted_iota(jnp.int32, (LANES,), 0)
        segmax = cv_v[0]
        off_vecs = [cv_v[1 + g] for g in range(NUM_GROUPS)]
        slot_mask = lane < SLOTS

        def fetch_x(i, p):
            return pltpu.make_async_copy(
                xf_hbm.at[pl.ds(w0 + i * C, C)], x_v.at[p], xsems(p))

        def compute_idx(p):
            def idx_body(t, _):
                for g in range(NUM_GROUPS):
                    xg = x_v[p, t, pl.ds(g * GROUP_SIZE, GROUP_SIZE)]
                    iv = jnp.minimum(jnp.maximum(xg.astype(jnp.int32), 0), segmax)
                    plsc.store_scatter(idx_v,
                                       [jnp.full((LANES,), p, jnp.int32),
                                        jnp.full((LANES,), g, jnp.int32),
                                        lane * C + t],
                                       iv + off_vecs[g], mask=slot_mask)
                return 0
            lax.fori_loop(0, C, idx_body, 0)

        def gather_cps(p):
            return [pltpu.make_async_copy(
                        table_hbm.at[idx_v.at[p, g]],
                        rows_v.at[p, pl.ds(g * IDX_PER_REF, IDX_PER_REF)],
                        gsems(p))
                    for g in range(NUM_GROUPS)]

        def fire_chunk(i, p):
            for cp in gather_cps(p):
                cp.start()
            pltpu.make_async_copy(
                ynum_hbm.at[pl.ds(w0 + i * C, C)], yp_v.at[p], ysems(p)).start()

        def wait_chunk(p):
            for cp in gather_cps(p):
                cp.wait()
            pltpu.make_async_copy(
                ynum_hbm.at[pl.ds(0, C)], yp_v.at[p], ysems(p)).wait()

        def compute_chunk(p):
            def tok_body(t, _):
                for g in range(NUM_GROUPS):
                    y0 = yp_v[p, t, pl.ds(g * 32, LANES)]
                    y1 = yp_v[p, t, pl.ds(g * 32 + LANES, LANES)]
                    acc = [lax.bitcast_convert_type(y0 << 16, jnp.float32),
                           lax.bitcast_convert_type(y1 << 16, jnp.float32),
                           lax.bitcast_convert_type(
                               y0 & jnp.int32(-65536), jnp.float32),
                           lax.bitcast_convert_type(
                               y1 & jnp.int32(-65536), jnp.float32)]
                    for s in range(SLOTS):
                        r = (g * SLOTS + s) * C + t
                        for h in range(2):
                            v = rows_v[p, r, pl.ds(h * LANES, LANES)]
                            acc[h] = acc[h] + lax.bitcast_convert_type(
                                v << 16, jnp.float32)
                            acc[2 + h] = acc[2 + h] + lax.bitcast_convert_type(
                                v & jnp.int32(-65536), jnp.float32)
                    for c in range(4):
                        out_v[p, t, pl.ds(g * OUT_DIM + c * LANES, LANES)] = (
                            jnp.maximum(acc[c], 0.0))
                return 0
            lax.fori_loop(0, C, tok_body, 0)

        def flush_chunk(i, p):
            pltpu.make_async_copy(out_v.at[p], out_hbm.at[pl.ds(w0 + i * C, C)],
                                  osems(p)).start()

        def drain_out(p):
            pltpu.make_async_copy(out_v.at[p], out_hbm.at[pl.ds(0, C)],
                                  osems(p)).wait()

        cp0 = fetch_x(0, 0)
        cp0.start()
        cp0.wait()
        compute_idx(0)
        fire_chunk(0, 0)
        fetch_x(1, 1).start()

        def loop_body(i, _):
            p = lax.rem(i, 2)
            pltpu.make_async_copy(
                xf_hbm.at[pl.ds(0, C)], x_v.at[p], xsems(p)).wait()
            compute_idx(p)

            @pl.when(i >= 2)
            def _():
                drain_out(p)
            fire_chunk(i, p)

            @pl.when(i + 1 < n_chunks)
            def _():
                fetch_x(i + 1, 1 - p).start()

            wait_chunk(1 - p)
            compute_chunk(1 - p)
            flush_chunk(i - 1, 1 - p)
            return 0

        lax.fori_loop(1, n_chunks, loop_body, 0)

        pl_last = (n_chunks - 1) % 2
        wait_chunk(pl_last)
        compute_chunk(pl_last)
        flush_chunk(n_chunks - 1, pl_last)
        drain_out(0)
        drain_out(1)

    return run(flat_table, xf, ynum, consts)


def _make_consts():
    consts = np.zeros((1 + NUM_GROUPS, LANES), np.int32)
    consts[0, :SLOTS] = [n - 1 for n in SEG_N]
    for g in range(NUM_GROUPS):
        consts[1 + g, :SLOTS] = [g * GSTRIDE + o for o in SEG_OFF]
    return jnp.asarray(consts)


def _make_bd_selector():
    s = np.zeros((F_IN, NUM_GROUPS, 9), np.float32)
    for g in range(NUM_GROUPS):
        for k in range(9):
            s[g * GROUP_SIZE + 7 + k, g, k] = 1.0
    return jnp.asarray(s)


def kernel(x, ability_emb, item_emb, species_emb, move_emb, W, b):
    B, S, F = x.shape
    xf = x.reshape(B * S, F)

    sp_p = jnp.pad(species_emb, ((0, 4), (0, 0)))
    ab_p = jnp.pad(ability_emb, ((0, 4), (0, 0)))
    it_p = jnp.pad(item_emb, ((0, 4), (0, 0)))
    w_pad = jnp.pad(W, ((0, 0), (0, 0), (0, 7)))

    bd = jnp.einsum("fgk,gok->fgo", _make_bd_selector(),
                    W[:, :, 128:137]).reshape(F_IN, F_OUT)

    flat_table = _build_flat_table(sp_p, ab_p, it_p, move_emb, w_pad)
    ynum = _build_ynum(xf, bd, b.reshape(1, F_OUT))
    out = _sc_encode(flat_table, xf, ynum, _make_consts())
    return out.reshape(B, S, F_OUT)
